# Initial kernel scaffold; baseline (speedup 1.0000x reference)
import jax
import jax.numpy as jnp
from jax.experimental import pallas as pl


def kernel(x, action, edge_index, W_emb, b_emb, Wg, att_src, att_dst, bg, Wf1, bf1, Wf2, bf2, bn_g, bn_b, Wd1, bd1, Wd2, bd2, Wd3, bd3):
    raise NotImplementedError("write your pallas kernel here")


# speedup vs baseline: 45890.6042x; 45890.6042x over previous
import jax
import jax.numpy as jnp
from jax.experimental import pallas as pl


def _bf(a):
    return a.astype(jnp.bfloat16)


def _decoder_body(xm_ref, act_ref, w1a_ref, w1b_ref, b1_ref,
                  w2_ref, b2_ref, w3_ref, b3_ref, o_ref):
    xm = xm_ref[...]
    act = act_ref[...]
    z = jnp.dot(_bf(xm), _bf(w1a_ref[...]), preferred_element_type=jnp.float32)
    actp = _bf(act).astype(jnp.float32).reshape(3, 1)
    w1bp = _bf(w1b_ref[...]).astype(jnp.float32)
    z = z + jnp.sum(actp * w1bp, axis=0, keepdims=True)
    z = jnp.maximum(z + b1_ref[...], 0.0)
    z = jnp.dot(_bf(z), _bf(w2_ref[...]), preferred_element_type=jnp.float32)
    z = jnp.maximum(z + b2_ref[...], 0.0)
    z = jnp.dot(_bf(z), _bf(w3_ref[...]), preferred_element_type=jnp.float32)
    o_ref[...] = z + b3_ref[...]


def kernel(x, action, edge_index, W_emb, b_emb, Wg, att_src, att_dst, bg,
           Wf1, bf1, Wf2, bf2, bn_g, bn_b, Wd1, bd1, Wd2, bd2, Wd3, bd3):
    xm = bn_b[-1].reshape(1, -1)
    out = pl.pallas_call(
        _decoder_body,
        out_shape=jax.ShapeDtypeStruct((1, 1), jnp.float32),
    )(
        xm,
        action.reshape(1, -1),
        Wd1[:128],
        Wd1[128:],
        bd1.reshape(1, -1),
        Wd2,
        bd2.reshape(1, -1),
        Wd3,
        bd3.reshape(1, -1),
    )
    return out.reshape(1)
